# Initial kernel scaffold; baseline (speedup 1.0000x reference)
import jax
import jax.numpy as jnp
from jax.experimental import pallas as pl


def kernel(x, h, node_mask, edge_mask, params):
    raise NotImplementedError("write your pallas kernel here")


# speedup vs baseline: 17.5942x; 17.5942x over previous
import jax
import jax.numpy as jnp
from jax.experimental import pallas as pl

N = 48
HID = 64
NL = 4


def _silu(v):
    return v / (1.0 + jnp.exp(-v))


def _egnn_body(x_ref, h_ref, nm_ref, em_ref,
               emb_W, emb_b, out_W, out_b,
               eW1a, eW1b, eW1c, eb1, eW2, eb2,
               nW1h, nW1a, nb1, nW2, nb2,
               cW1, cb1, cW2T,
               o_ref):
    f32 = jnp.float32
    nm = nm_ref[0]
    em = em_ref[0]
    C = x_ref[0] * nm
    H = h_ref[0] * nm
    H = jnp.dot(H, emb_W[:], preferred_element_type=f32) + emb_b[:]
    ones = jnp.ones((N, 1), f32)

    for l in range(NL):
        sq = jnp.sum(C * C, axis=1, keepdims=True)
        P = jnp.concatenate([C, sq, ones], axis=1)
        Q = jnp.concatenate([-2.0 * C, ones, sq], axis=1)
        R = jnp.maximum(
            jax.lax.dot_general(P, Q, (((1,), (1,)), ((), ())),
                                preferred_element_type=f32), 0.0)
        A = jnp.dot(H, eW1a[l], preferred_element_type=f32) + eb1[l]
        B = jnp.dot(H, eW1b[l], preferred_element_type=f32)
        ef = _silu(A[:, None, :] + B[None, :, :]
                   + R[:, :, None] * eW1c[l][None])
        ef2 = _silu(
            jnp.dot(ef.reshape(N * N, HID), eW2[l],
                    preferred_element_type=f32) + eb2[l])
        ef2 = (ef2.reshape(N, N, HID) * em[:, :, None]).reshape(N * N, HID)
        t = _silu(jnp.dot(ef2, cW1[l], preferred_element_type=f32) + cb1[l])
        phi = jnp.sum(t.reshape(N, N, HID) * cW2T[l][None], axis=2)
        S = (phi * em) / (jnp.sqrt(R) + 1.0)
        rs = jnp.sum(S, axis=1, keepdims=True)
        C = C + C * rs - jnp.dot(S, C, preferred_element_type=f32)
        agg = jnp.sum(ef2.reshape(N, N, HID), axis=1)
        o = _silu(jnp.dot(H, nW1h[l], preferred_element_type=f32)
                  + jnp.dot(agg, nW1a[l], preferred_element_type=f32)
                  + nb1[l])
        o = jnp.dot(o, nW2[l], preferred_element_type=f32) + nb2[l]
        H = (H + o) * nm

    Ho = jnp.dot(H, out_W[:], preferred_element_type=f32) + out_b[:]
    o_ref[0] = Ho * nm


def kernel(x, h, node_mask, edge_mask, params):
    bs, n, dims = x.shape
    in_nf = h.shape[-1]
    out_nf = params['out_W'].shape[1]

    em = edge_mask.reshape(bs, n, n)
    L = params['layers']
    eW1 = jnp.stack([Li['e_W1'] for Li in L])
    eW1a = eW1[:, :HID, :]
    eW1b = eW1[:, HID:2 * HID, :]
    eW1c = eW1[:, 2 * HID, :]
    eb1 = jnp.stack([Li['e_b1'] for Li in L])
    eW2 = jnp.stack([Li['e_W2'] for Li in L])
    eb2 = jnp.stack([Li['e_b2'] for Li in L])
    nW1 = jnp.stack([Li['n_W1'] for Li in L])
    nW1h = nW1[:, :HID, :]
    nW1a = nW1[:, HID:, :]
    nb1 = jnp.stack([Li['n_b1'] for Li in L])
    nW2 = jnp.stack([Li['n_W2'] for Li in L])
    nb2 = jnp.stack([Li['n_b2'] for Li in L])
    cW1 = jnp.stack([Li['c_W1'] for Li in L])
    cb1 = jnp.stack([Li['c_b1'] for Li in L])
    cW2T = jnp.stack([Li['c_W2'] for Li in L])[:, :, 0]

    def full(a):
        return pl.BlockSpec(a.shape, lambda b: (0,) * a.ndim)

    weights = [params['emb_W'], params['emb_b'], params['out_W'],
               params['out_b'], eW1a, eW1b, eW1c, eb1, eW2, eb2,
               nW1h, nW1a, nb1, nW2, nb2, cW1, cb1, cW2T]

    out = pl.pallas_call(
        _egnn_body,
        grid=(bs,),
        in_specs=[
            pl.BlockSpec((1, n, dims), lambda b: (b, 0, 0)),
            pl.BlockSpec((1, n, in_nf), lambda b: (b, 0, 0)),
            pl.BlockSpec((1, n, 1), lambda b: (b, 0, 0)),
            pl.BlockSpec((1, n, n), lambda b: (b, 0, 0)),
        ] + [full(w) for w in weights],
        out_specs=pl.BlockSpec((1, n, out_nf), lambda b: (b, 0, 0)),
        out_shape=jax.ShapeDtypeStruct((bs, n, out_nf), jnp.float32),
    )(x, h, node_mask, em, *weights)
    return out


# speedup vs baseline: 17.6410x; 1.0027x over previous
import jax
import jax.numpy as jnp
from jax.experimental import pallas as pl
from jax.experimental.pallas import tpu as pltpu

N = 48
HID = 64
NL = 4


def _silu(v):
    return v / (1.0 + jnp.exp(-v))


def _egnn_body(x_ref, h_ref, nm_ref, em_ref,
               emb_W, emb_b, out_W, out_b,
               eW1a, eW1b, eW1c, eb1, eW2, eb2,
               nW1h, nW1a, nb1, nW2, nb2,
               cW1, cb1, cW2T,
               o_ref):
    f32 = jnp.float32
    nm = nm_ref[0]
    em = em_ref[0]
    C = x_ref[0] * nm
    H = h_ref[0] * nm
    H = jnp.dot(H, emb_W[:], preferred_element_type=f32) + emb_b[:]
    ones = jnp.ones((N, 1), f32)

    for l in range(NL):
        sq = jnp.sum(C * C, axis=1, keepdims=True)
        P = jnp.concatenate([C, sq, ones], axis=1)
        Q = jnp.concatenate([-2.0 * C, ones, sq], axis=1)
        R = jnp.maximum(
            jax.lax.dot_general(P, Q, (((1,), (1,)), ((), ())),
                                preferred_element_type=f32), 0.0)
        A = jnp.dot(H, eW1a[l], preferred_element_type=f32) + eb1[l]
        B = jnp.dot(H, eW1b[l], preferred_element_type=f32)
        ef = _silu(A[:, None, :] + B[None, :, :]
                   + R[:, :, None] * eW1c[l][None])
        ef2 = _silu(
            jnp.dot(ef.reshape(N * N, HID), eW2[l],
                    preferred_element_type=f32) + eb2[l])
        ef2 = (ef2.reshape(N, N, HID) * em[:, :, None]).reshape(N * N, HID)
        t = _silu(jnp.dot(ef2, cW1[l], preferred_element_type=f32) + cb1[l])
        phi = jnp.sum(t.reshape(N, N, HID) * cW2T[l][None], axis=2)
        S = (phi * em) / (jnp.sqrt(R) + 1.0)
        rs = jnp.sum(S, axis=1, keepdims=True)
        C = C + C * rs - jnp.dot(S, C, preferred_element_type=f32)
        agg = jnp.sum(ef2.reshape(N, N, HID), axis=1)
        o = _silu(jnp.dot(H, nW1h[l], preferred_element_type=f32)
                  + jnp.dot(agg, nW1a[l], preferred_element_type=f32)
                  + nb1[l])
        o = jnp.dot(o, nW2[l], preferred_element_type=f32) + nb2[l]
        H = (H + o) * nm

    Ho = jnp.dot(H, out_W[:], preferred_element_type=f32) + out_b[:]
    o_ref[0] = Ho * nm


def kernel(x, h, node_mask, edge_mask, params):
    bs, n, dims = x.shape
    in_nf = h.shape[-1]
    out_nf = params['out_W'].shape[1]

    em = edge_mask.reshape(bs, n, n)
    L = params['layers']
    eW1 = jnp.stack([Li['e_W1'] for Li in L])
    eW1a = eW1[:, :HID, :]
    eW1b = eW1[:, HID:2 * HID, :]
    eW1c = eW1[:, 2 * HID, :]
    eb1 = jnp.stack([Li['e_b1'] for Li in L])
    eW2 = jnp.stack([Li['e_W2'] for Li in L])
    eb2 = jnp.stack([Li['e_b2'] for Li in L])
    nW1 = jnp.stack([Li['n_W1'] for Li in L])
    nW1h = nW1[:, :HID, :]
    nW1a = nW1[:, HID:, :]
    nb1 = jnp.stack([Li['n_b1'] for Li in L])
    nW2 = jnp.stack([Li['n_W2'] for Li in L])
    nb2 = jnp.stack([Li['n_b2'] for Li in L])
    cW1 = jnp.stack([Li['c_W1'] for Li in L])
    cb1 = jnp.stack([Li['c_b1'] for Li in L])
    cW2T = jnp.stack([Li['c_W2'] for Li in L])[:, :, 0]

    def full(a):
        return pl.BlockSpec(a.shape, lambda b: (0,) * a.ndim)

    weights = [params['emb_W'], params['emb_b'], params['out_W'],
               params['out_b'], eW1a, eW1b, eW1c, eb1, eW2, eb2,
               nW1h, nW1a, nb1, nW2, nb2, cW1, cb1, cW2T]

    out = pl.pallas_call(
        _egnn_body,
        grid=(bs,),
        in_specs=[
            pl.BlockSpec((1, n, dims), lambda b: (b, 0, 0)),
            pl.BlockSpec((1, n, in_nf), lambda b: (b, 0, 0)),
            pl.BlockSpec((1, n, 1), lambda b: (b, 0, 0)),
            pl.BlockSpec((1, n, n), lambda b: (b, 0, 0)),
        ] + [full(w) for w in weights],
        out_specs=pl.BlockSpec((1, n, out_nf), lambda b: (b, 0, 0)),
        out_shape=jax.ShapeDtypeStruct((bs, n, out_nf), jnp.float32),
        compiler_params=pltpu.CompilerParams(
            dimension_semantics=("parallel",)),
    )(x, h, node_mask, em, *weights)
    return out


# speedup vs baseline: 22.5497x; 1.2783x over previous
import jax
import jax.numpy as jnp
from jax.experimental import pallas as pl
from jax.experimental.pallas import tpu as pltpu

N = 48
HID = 64
NL = 4
SB = 4


def _silu(v):
    return v * (0.5 * jnp.tanh(v * 0.5) + 0.5)


def _egnn_body(x_ref, h_ref, nm_ref, em_ref,
               emb_W, emb_b, out_W, out_b,
               eW1a, eW1b, eW1c, eb1, eW2, eb2,
               nW1h, nW1a, nb1, nW2, nb2,
               cW1, cb1, cW2T,
               o_ref):
    f32 = jnp.float32
    ones = jnp.ones((N, 1), f32)
    nm = nm_ref[...]
    nmf = nm.reshape(SB * N, 1)
    em = em_ref[...]
    Cs = [x_ref[s] * nm[s] for s in range(SB)]
    H = (h_ref[...] * nm).reshape(SB * N, -1)
    H = jnp.dot(H, emb_W[:], preferred_element_type=f32) + emb_b[:]

    for l in range(NL):
        Rs = []
        for s in range(SB):
            C = Cs[s]
            sq = jnp.sum(C * C, axis=1, keepdims=True)
            P = jnp.concatenate([C, sq, ones], axis=1)
            Q = jnp.concatenate([-2.0 * C, ones, sq], axis=1)
            Rs.append(jax.lax.dot_general(
                P, Q, (((1,), (1,)), ((), ())), preferred_element_type=f32))
        R = jnp.maximum(jnp.stack(Rs), 0.0)
        A = (jnp.dot(H, eW1a[l], preferred_element_type=f32)
             + eb1[l]).reshape(SB, N, HID)
        B = jnp.dot(H, eW1b[l], preferred_element_type=f32).reshape(SB, N, HID)
        ef = _silu(A[:, :, None, :] + B[:, None, :, :]
                   + R[..., None] * eW1c[l])
        ef2 = _silu(
            jnp.dot(ef.reshape(SB * N * N, HID), eW2[l],
                    preferred_element_type=f32) + eb2[l])
        ef2 = ef2.reshape(SB, N, N, HID) * em[..., None]
        t = _silu(jnp.dot(ef2.reshape(SB * N * N, HID), cW1[l],
                          preferred_element_type=f32) + cb1[l])
        phi = jnp.sum(t.reshape(SB, N, N, HID) * cW2T[l], axis=3)
        S = (phi * em) / (jnp.sqrt(R) + 1.0)
        rs = jnp.sum(S, axis=2, keepdims=True)
        for s in range(SB):
            Cs[s] = (Cs[s] + Cs[s] * rs[s]
                     - jnp.dot(S[s], Cs[s], preferred_element_type=f32))
        agg = jnp.sum(ef2, axis=2).reshape(SB * N, HID)
        o = _silu(jnp.dot(H, nW1h[l], preferred_element_type=f32)
                  + jnp.dot(agg, nW1a[l], preferred_element_type=f32)
                  + nb1[l])
        o = jnp.dot(o, nW2[l], preferred_element_type=f32) + nb2[l]
        H = (H + o) * nmf

    Ho = jnp.dot(H, out_W[:], preferred_element_type=f32) + out_b[:]
    o_ref[...] = (Ho * nmf).reshape(SB, N, -1)


def kernel(x, h, node_mask, edge_mask, params):
    bs, n, dims = x.shape
    in_nf = h.shape[-1]
    out_nf = params['out_W'].shape[1]

    em = edge_mask.reshape(bs, n, n)
    L = params['layers']
    eW1 = jnp.stack([Li['e_W1'] for Li in L])
    eW1a = eW1[:, :HID, :]
    eW1b = eW1[:, HID:2 * HID, :]
    eW1c = eW1[:, 2 * HID, :]
    eb1 = jnp.stack([Li['e_b1'] for Li in L])
    eW2 = jnp.stack([Li['e_W2'] for Li in L])
    eb2 = jnp.stack([Li['e_b2'] for Li in L])
    nW1 = jnp.stack([Li['n_W1'] for Li in L])
    nW1h = nW1[:, :HID, :]
    nW1a = nW1[:, HID:, :]
    nb1 = jnp.stack([Li['n_b1'] for Li in L])
    nW2 = jnp.stack([Li['n_W2'] for Li in L])
    nb2 = jnp.stack([Li['n_b2'] for Li in L])
    cW1 = jnp.stack([Li['c_W1'] for Li in L])
    cb1 = jnp.stack([Li['c_b1'] for Li in L])
    cW2T = jnp.stack([Li['c_W2'] for Li in L])[:, :, 0]

    def full(a):
        return pl.BlockSpec(a.shape, lambda b: (0,) * a.ndim)

    weights = [params['emb_W'], params['emb_b'], params['out_W'],
               params['out_b'], eW1a, eW1b, eW1c, eb1, eW2, eb2,
               nW1h, nW1a, nb1, nW2, nb2, cW1, cb1, cW2T]

    out = pl.pallas_call(
        _egnn_body,
        grid=(bs // SB,),
        in_specs=[
            pl.BlockSpec((SB, n, dims), lambda b: (b, 0, 0)),
            pl.BlockSpec((SB, n, in_nf), lambda b: (b, 0, 0)),
            pl.BlockSpec((SB, n, 1), lambda b: (b, 0, 0)),
            pl.BlockSpec((SB, n, n), lambda b: (b, 0, 0)),
        ] + [full(w) for w in weights],
        out_specs=pl.BlockSpec((SB, n, out_nf), lambda b: (b, 0, 0)),
        out_shape=jax.ShapeDtypeStruct((bs, n, out_nf), jnp.float32),
        compiler_params=pltpu.CompilerParams(
            dimension_semantics=("parallel",)),
    )(x, h, node_mask, em, *weights)
    return out
